# baseline (speedup 1.0000x reference)
import functools

import jax
import jax.numpy as jnp
from jax import lax
from jax.experimental import pallas as pl
from jax.experimental.pallas import tpu as pltpu
from jax.experimental.pallas import tpu_sc as plsc

DT = 1.0
_NC, _NS, _L = 2, 16, 16
_NW = _NC * _NS


def _lstm_tc(B, N, T, F, H, HP,
             xFT_ref, WallT_ref, b_ref, out_ref):
    BN = B * N
    WallT = WallT_ref[...]
    b = b_ref[...]

    C = 16
    CH = T // C

    def sigm(v):
        return 0.5 * jnp.tanh(0.5 * v) + 0.5

    NL = 2
    LB = BN // NL
    WhhT = WallT[:, 0:H]
    WihT = WallT[:, H:H + F]

    def xproj(t, j):
        xtT = xFT_ref[:, pl.ds(t * BN + j * LB, LB)]
        return b + jnp.dot(WihT, xtT, preferred_element_type=jnp.float32)

    def outer(ci, carry):
        hs, cs, xw = list(carry[0]), list(carry[1]), list(carry[2])
        for k in range(C):
            t = ci * C + k
            for j in range(NL):
                gT = xw[j] + jnp.dot(WhhT, hs[j],
                                     preferred_element_type=jnp.float32)
                xw[j] = xproj(jnp.minimum(t + 1, T - 1), j)
                i = sigm(gT[0:H])
                f = sigm(gT[H:2 * H])
                g = jnp.tanh(gT[2 * H:3 * H])
                o = sigm(gT[3 * H:4 * H])
                cs[j] = f * cs[j] + i * g
                hs[j] = o * jnp.tanh(cs[j])
        return (tuple(hs), tuple(cs), tuple(xw))

    h0 = tuple(jnp.zeros((H, LB), jnp.float32) for _ in range(NL))
    c0 = tuple(jnp.zeros((H, LB), jnp.float32) for _ in range(NL))
    xw0 = tuple(xproj(0, j) for j in range(NL))
    hs, _, _ = jax.lax.fori_loop(0, CH, outer, (h0, c0, xw0))
    hT = jnp.concatenate(hs, axis=1)
    out_ref[:, 0:H] = hT.T
    out_ref[:, H:HP] = jnp.zeros((B * N, HP - H), jnp.float32)


def _sc_aggregate(BN, H, EP):
    epw = EP // _NW
    rps = BN // _NS
    mesh = plsc.VectorSubcoreMesh(core_axis_name="c", subcore_axis_name="s")

    @functools.partial(
        pl.kernel,
        out_type=jax.ShapeDtypeStruct((2 * BN, H), jnp.float32),
        mesh=mesh,
        scratch_types=[
            pltpu.VMEM((epw,), jnp.int32),
            pltpu.VMEM((epw,), jnp.int32),
            pltpu.VMEM((epw, H), jnp.float32),
            pltpu.VMEM((epw, H), jnp.float32),
            pltpu.VMEM((epw, H), jnp.float32),
            pltpu.VMEM((rps, H), jnp.float32),
            pltpu.VMEM_SHARED((BN, H), jnp.float32),
            pltpu.SemaphoreType.DMA,
        ],
    )
    def agg_kernel(h_hbm, src_hbm, dst_hbm, wexp_hbm, out_hbm,
                   src_v, dst_v, w_v, rows_v, msg_v, zero_v, acc_sh, sem):
        cid = lax.axis_index("c")
        sid = lax.axis_index("s")
        wid = sid * _NC + cid
        ebase = wid * epw
        rbase = sid * rps

        zvec = jnp.zeros((_L,), jnp.float32)
        for r in range(rps):
            for v in range(H // _L):
                zero_v[r, pl.ds(v * _L, _L)] = zvec
        pltpu.sync_copy(zero_v, acc_sh.at[pl.ds(rbase, rps)])

        pltpu.sync_copy(src_hbm.at[pl.ds(ebase, epw)], src_v)
        pltpu.sync_copy(dst_hbm.at[pl.ds(ebase, epw)], dst_v)
        pltpu.sync_copy(wexp_hbm.at[pl.ds(ebase, epw)], w_v)
        pltpu.async_copy(h_hbm.at[src_v], rows_v, sem).wait()
        for e in range(epw):
            for v in range(H // _L):
                sl = pl.ds(v * _L, _L)
                msg_v[e, sl] = rows_v[e, sl] * w_v[e, sl]

        plsc.subcore_barrier()
        pltpu.sync_copy(msg_v, acc_sh.at[dst_v], add=True)
        plsc.subcore_barrier()
        pltpu.sync_copy(acc_sh.at[pl.ds(rbase, rps)],
                        out_hbm.at[pl.ds(cid * BN + rbase, rps)])

    return agg_kernel


def _dense_layer(B, N, H, HP, h_ref, agg_ref, Ws_ref, Wm_ref, bl_ref,
                 out_ref):
    h = h_ref[:, 0:H]
    BN = B * N
    agg = agg_ref[0:BN, 0:H] + agg_ref[BN:2 * BN, 0:H]
    z = (jnp.dot(h, Ws_ref[...], preferred_element_type=jnp.float32)
         + jnp.dot(agg, Wm_ref[...],
                   preferred_element_type=jnp.float32) + bl_ref[...])
    out_ref[:, 0:H] = jax.nn.gelu(z) + h
    out_ref[:, H:HP] = jnp.zeros((BN, HP - H), jnp.float32)


def _dense_head(B, N, H, HOR,
                h_ref, agg_ref, Ws_ref, Wm_ref, bl_ref,
                hW1_ref, hb1_ref, hW2_ref, hb2_ref, out_ref):
    h = h_ref[:, 0:H]
    BN = B * N
    agg = agg_ref[0:BN, 0:H] + agg_ref[BN:2 * BN, 0:H]
    z = (jnp.dot(h, Ws_ref[...], preferred_element_type=jnp.float32)
         + jnp.dot(agg, Wm_ref[...],
                   preferred_element_type=jnp.float32) + bl_ref[...])
    h = jax.nn.gelu(z) + h
    z = jax.nn.gelu(jnp.dot(h, hW1_ref[...], preferred_element_type=jnp.float32)
                    + hb1_ref[...])
    out_ref[...] = (jnp.dot(z, hW2_ref[...], preferred_element_type=jnp.float32)
                    + hb2_ref[...])


def kernel(x, Wih, Whh, b_lstm, K, X, Ws0, Wm0, bl0, Ws1, Wm1, bl1,
           hW1, hb1, hW2, hb2, edge_index):
    B, N, T, F = x.shape
    H = Whh.shape[0]
    HOR = hW2.shape[1]
    E = edge_index.shape[1]
    BN = B * N

    xFT = jnp.transpose(x.reshape(BN, T, F), (2, 1, 0)).reshape(F, T * BN)
    WallT = jnp.concatenate([Whh, Wih], axis=0).T
    HP = 128
    h = pl.pallas_call(
        functools.partial(_lstm_tc, B, N, T, F, H, HP),
        out_shape=jax.ShapeDtypeStruct((BN, HP), jnp.float32),
    )(xFT, WallT, b_lstm.reshape(4 * H, 1))

    denom = K - K * X + 0.5 * DT
    w = ((-K * X + 0.5 * DT) / denom) + ((K * X + 0.5 * DT) / denom)
    offs = (jnp.arange(B, dtype=jnp.int32) * N)[:, None]
    EP = ((B * E + _NW * 8 - 1) // (_NW * 8)) * (_NW * 8)
    pad = EP - B * E
    bigsrc = jnp.pad((edge_index[0][None, :] + offs).reshape(-1), (0, pad))
    bigdst = jnp.pad((edge_index[1][None, :] + offs).reshape(-1), (0, pad))
    wexp = jnp.pad(jnp.broadcast_to(jnp.tile(w, B)[:, None], (B * E, HP)),
                   ((0, pad), (0, 0)))

    agg_fn = _sc_aggregate(BN, HP, EP)
    dense_fn = pl.pallas_call(
        functools.partial(_dense_layer, B, N, H, HP),
        out_shape=jax.ShapeDtypeStruct((BN, HP), jnp.float32),
    )

    agg0 = agg_fn(h, bigsrc, bigdst, wexp)
    h1 = dense_fn(h, agg0, Ws0, Wm0, bl0.reshape(1, H))

    agg1 = agg_fn(h1, bigsrc, bigdst, wexp)
    out = pl.pallas_call(
        functools.partial(_dense_head, B, N, H, HOR),
        out_shape=jax.ShapeDtypeStruct((BN, HOR), jnp.float32),
    )(h1, agg1, Ws1, Wm1, bl1.reshape(1, H), hW1, hb1.reshape(1, H),
      hW2, hb2.reshape(1, HOR))
    return out.reshape(B, N, HOR)
